# baseline (speedup 1.0000x reference)
import functools

import jax
import jax.numpy as jnp
from jax import lax
from jax.experimental import pallas as pl
from jax.experimental.pallas import tpu as pltpu
from jax.experimental.pallas import tpu_sc as plsc

EMBED = 64
LANES = 16
NC, NS = 2, 16
NW = NC * NS
BATCH = 4096
SEQ = 200
TOTAL = BATCH * SEQ
BPW = BATCH // NW
PER_W = BPW * SEQ
ER = EMBED // 8
TILE = 8 * 128


@functools.partial(
    pl.kernel,
    mesh=plsc.VectorSubcoreMesh(core_axis_name="c", subcore_axis_name="s"),
    out_type=jax.ShapeDtypeStruct((SEQ, ER, NW, TILE), jnp.float32),
    scratch_types=[
        pltpu.VMEM((PER_W,), jnp.int32),
        pltpu.VMEM((PER_W,), jnp.int32),
        pltpu.VMEM((2, BPW, EMBED), jnp.float32),
        pltpu.VMEM((2, EMBED * BPW), jnp.float32),
        pltpu.SemaphoreType.DMA,
        pltpu.SemaphoreType.DMA,
        pltpu.SemaphoreType.DMA,
        pltpu.SemaphoreType.DMA,
    ],
    compiler_params=pltpu.CompilerParams(
        use_tc_tiling_on_sc=False, needs_layout_passes=False),
)
def _sc_gather(table_hbm, idx_hbm, out_hbm, idx_v, idxt_v, rows_v, tiles_v,
               g0, g1, o0, o1):
    gsem = (g0, g1)
    osem = (o0, o1)
    wid = lax.axis_index("s") * NC + lax.axis_index("c")
    base = wid * PER_W
    pltpu.sync_copy(idx_hbm.at[pl.ds(base, PER_W)], idx_v)

    iota = lax.iota(jnp.int32, LANES)
    pre_s = [(16 * g + iota) * SEQ for g in range(BPW // LANES)]

    def build_idxt(s, carry):
        for g in range(BPW // LANES):
            val = plsc.load_gather(idx_v, [pre_s[g] + s])
            idxt_v[pl.ds(s * BPW + 16 * g, LANES)] = val
        return carry

    lax.fori_loop(0, SEQ, build_idxt, 0)

    def fire_g(s, b):
        pltpu.async_copy(
            table_hbm.at[idxt_v.at[pl.ds(s * BPW, BPW)]],
            rows_v.at[b],
            gsem[b],
        )

    def wait_g(b):
        pltpu.make_async_copy(
            table_hbm.at[pl.ds(0, BPW), :],
            rows_v.at[b],
            gsem[b],
        ).wait()

    def transpose(b):
        def egrp(eg, carry):
            for eo in range(8):
                e = eg * 8 + eo
                col = jnp.full((LANES,), e, jnp.int32)
                for g in range(BPW // LANES):
                    val = plsc.load_gather(
                        rows_v.at[b], [16 * g + iota, col])
                    tiles_v[b, pl.ds(e * BPW + 16 * g, LANES)] = val
            return carry

        lax.fori_loop(0, EMBED // 8, egrp, 0)

    def fire_o(s, b):
        for er in range(ER):
            pltpu.async_copy(
                tiles_v.at[b, pl.ds(er * TILE, TILE)],
                out_hbm.at[s, er, wid],
                osem[b],
            )

    def wait_o(b):
        for er in range(ER):
            pltpu.make_async_copy(
                tiles_v.at[b, pl.ds(er * TILE, TILE)],
                out_hbm.at[0, er, 0],
                osem[b],
            ).wait()

    fire_g(0, 0)
    fire_g(1, 1)
    for u in (0, 1):
        wait_g(u)
        transpose(u)
        fire_o(u, u)
        fire_g(u + 2, u)

    def unit(u, b):
        wait_g(b)
        wait_o(b)
        transpose(b)
        fire_o(u, b)
        fire_g(u + 2, b)
        return

    def pair(p, carry):
        u = 2 + 2 * p
        unit(u, 0)
        unit(u + 1, 1)
        return carry

    lax.fori_loop(0, (SEQ - 4) // 2, pair, 0)
    for u in (SEQ - 2, SEQ - 1):
        b = u % 2
        wait_g(b)
        wait_o(b)
        transpose(b)
        fire_o(u, b)
    for b in (0, 1):
        wait_o(b)


def kernel(tokens, wte_weight):
    idx1d = tokens.astype(jnp.int32).reshape(TOTAL)
    out4 = _sc_gather(wte_weight, idx1d)
    o5 = out4.reshape(SEQ, ER, NW, 8, 128)
    return o5.transpose(2, 4, 0, 1, 3).reshape(BATCH, SEQ, EMBED)


# speedup vs baseline: 1.5145x; 1.5145x over previous
import functools

import jax
import jax.numpy as jnp
from jax import lax
from jax.experimental import pallas as pl
from jax.experimental.pallas import tpu as pltpu
from jax.experimental.pallas import tpu_sc as plsc

EMBED = 64
LANES = 16
NC, NS = 2, 16
NW = NC * NS
BATCH = 4096
SEQ = 200
TOTAL = BATCH * SEQ
BPW = BATCH // NW
PER_W = BPW * SEQ
ER = EMBED // 8
TILE = 8 * 128


@functools.partial(
    pl.kernel,
    mesh=plsc.VectorSubcoreMesh(core_axis_name="c", subcore_axis_name="s"),
    out_type=jax.ShapeDtypeStruct((SEQ, ER, NW, TILE), jnp.float32),
    scratch_types=[
        pltpu.VMEM((PER_W,), jnp.int32),
        pltpu.VMEM((PER_W,), jnp.int32),
        pltpu.VMEM((2, BPW, EMBED), jnp.float32),
        pltpu.VMEM((2, ER, TILE), jnp.float32),
        pltpu.SemaphoreType.DMA,
        pltpu.SemaphoreType.DMA,
        pltpu.SemaphoreType.DMA,
        pltpu.SemaphoreType.DMA,
    ],
    compiler_params=pltpu.CompilerParams(
        use_tc_tiling_on_sc=False, needs_layout_passes=False),
)
def _sc_gather(table_hbm, idx_hbm, out_hbm, idx_v, idxt_v, rows_v, tiles_v,
               g0, g1, o0, o1):
    gsem = (g0, g1)
    osem = (o0, o1)
    wid = lax.axis_index("s") * NC + lax.axis_index("c")
    base = wid * PER_W
    pltpu.sync_copy(idx_hbm.at[pl.ds(base, PER_W)], idx_v)

    iota = lax.iota(jnp.int32, LANES)
    pre_s = [(16 * g + iota) * SEQ for g in range(BPW // LANES)]

    @plsc.parallel_loop(0, SEQ, unroll=2)
    def _build_idxt(s):
        for g in range(BPW // LANES):
            val = plsc.load_gather(idx_v, [pre_s[g] + s])
            idxt_v[pl.ds(s * BPW + 16 * g, LANES)] = val

    def fire_g(s, b):
        pltpu.async_copy(
            table_hbm.at[idxt_v.at[pl.ds(s * BPW, BPW)]],
            rows_v.at[b],
            gsem[b],
        )

    def wait_g(b):
        pltpu.make_async_copy(
            table_hbm.at[pl.ds(0, BPW), :],
            rows_v.at[b],
            gsem[b],
        ).wait()

    def transpose(b):
        @plsc.parallel_loop(0, EMBED, unroll=4)
        def _t(e):
            er = e >> 3
            off = (e & 7) * 128
            col = jnp.full((LANES,), e, jnp.int32)
            for g in range(BPW // LANES):
                val = plsc.load_gather(rows_v.at[b], [16 * g + iota, col])
                tiles_v[b, er, pl.ds(off + 16 * g, LANES)] = val

    def fire_o(s, b):
        pltpu.async_copy(
            tiles_v.at[b],
            out_hbm.at[s, :, wid],
            osem[b],
        )

    def wait_o(b):
        pltpu.make_async_copy(
            tiles_v.at[b],
            out_hbm.at[0, :, 0],
            osem[b],
        ).wait()

    fire_g(0, 0)
    fire_g(1, 1)
    for u in (0, 1):
        wait_g(u)
        transpose(u)
        fire_o(u, u)
        fire_g(u + 2, u)

    def unit(u, b):
        wait_g(b)
        wait_o(b)
        transpose(b)
        fire_o(u, b)
        fire_g(u + 2, b)

    def pair(p, carry):
        u = 2 + 2 * p
        unit(u, 0)
        unit(u + 1, 1)
        return carry

    lax.fori_loop(0, (SEQ - 4) // 2, pair, 0)
    for u in (SEQ - 2, SEQ - 1):
        b = u % 2
        wait_g(b)
        wait_o(b)
        transpose(b)
        fire_o(u, b)
    for b in (0, 1):
        wait_o(b)


def kernel(tokens, wte_weight):
    idx1d = tokens.astype(jnp.int32).reshape(TOTAL)
    out4 = _sc_gather(wte_weight, idx1d)
    o5 = out4.reshape(SEQ, ER, NW, 8, 128)
    return o5.transpose(2, 4, 0, 1, 3).reshape(BATCH, SEQ, EMBED)
